# baseline (speedup 1.0000x reference)
import functools

import jax
import jax.numpy as jnp
from jax import lax
from jax.experimental import pallas as pl
from jax.experimental.pallas import tpu as pltpu
from jax.experimental.pallas import tpu_sc as plsc

B = 16384
F = 26
V = 100000
D = 16
ND = 13
H1 = 256
H2 = 128
FD = F * D

_NC = 2
_NS = 16
_NW = _NC * _NS
_NTASK = FD + F
_NROUND = -(-_NTASK // _NW)


_H = V // 2
_HB = B // 2


def _sc_gather_body(emb_hbm, lin_hbm, idx_hbm, fen_out, lin_out,
                    row_a, row_b, idx_v, res_v, sem_a, sem_b):
    wid = lax.axis_index("s") * _NC + lax.axis_index("c")

    def start_half(t, half, buf, sem):
        is_lin = t >= FD

        @pl.when(jnp.logical_not(is_lin))
        def _se():
            pltpu.make_async_copy(
                emb_hbm.at[pl.ds(t * V + half * _H, _H)], buf, sem).start()

        @pl.when(is_lin)
        def _sl():
            pltpu.make_async_copy(
                lin_hbm.at[pl.ds((t - FD) * V + half * _H, _H)], buf,
                sem).start()

    def wait_half(buf, sem):
        pltpu.make_async_copy(emb_hbm.at[pl.ds(0, _H)], buf, sem).wait()

    def pass0(hb):
        def step(i, c):
            ii = idx_v[pl.ds(hb * _HB + i * 16, 16)]
            pos = jnp.minimum(ii, _H - 1)
            res_v[pl.ds(i * 16, 16)] = plsc.load_gather(row_a, [pos])
            return c

        lax.fori_loop(0, _HB // 16, step, 0, unroll=4)

    def pass1(hb):
        def step(i, c):
            ii = idx_v[pl.ds(hb * _HB + i * 16, 16)]
            pos = jnp.maximum(ii - _H, 0)
            g1 = plsc.load_gather(row_b, [pos])
            prev = res_v[pl.ds(i * 16, 16)]
            res_v[pl.ds(i * 16, 16)] = jnp.where(ii < _H, prev, g1)
            return c

        lax.fori_loop(0, _HB // 16, step, 0, unroll=4)

    def store(t, is_lin, hb):
        @pl.when(jnp.logical_not(is_lin))
        def _of():
            pltpu.sync_copy(res_v, fen_out.at[pl.ds(t * B + hb * _HB, _HB)])

        @pl.when(is_lin)
        def _ol():
            pltpu.sync_copy(
                res_v, lin_out.at[pl.ds((t - FD) * B + hb * _HB, _HB)])

    t0 = jnp.minimum(wid, _NTASK - 1)
    start_half(t0, 0, row_a, sem_a)
    for j in range(_NROUND):
        t = jnp.minimum(wid + _NW * j, _NTASK - 1)
        is_lin = t >= FD
        f = jnp.where(is_lin, t - FD, t // D)
        start_half(t, 1, row_b, sem_b)
        pltpu.sync_copy(idx_hbm.at[pl.ds(f * B, B)], idx_v)
        wait_half(row_a, sem_a)
        pass0(0)
        wait_half(row_b, sem_b)
        pass1(0)
        store(t, is_lin, 0)
        pass0(1)
        if j + 1 < _NROUND:
            t_next = jnp.minimum(wid + _NW * (j + 1), _NTASK - 1)
            start_half(t_next, 0, row_a, sem_a)
        pass1(1)
        store(t, is_lin, 1)


def _sc_gather(emb_flat, lin_flat, idx_flat):
    call = pl.kernel(
        _sc_gather_body,
        out_type=(
            jax.ShapeDtypeStruct((FD * B,), jnp.float32),
            jax.ShapeDtypeStruct((F * B,), jnp.float32),
        ),
        mesh=plsc.VectorSubcoreMesh(core_axis_name="c", subcore_axis_name="s"),
        scratch_types=[
            pltpu.VMEM((_H,), jnp.float32),
            pltpu.VMEM((_H,), jnp.float32),
            pltpu.VMEM((B,), jnp.int32),
            pltpu.VMEM((_HB,), jnp.float32),
            pltpu.SemaphoreType.DMA,
            pltpu.SemaphoreType.DMA,
        ],
        compiler_params=pltpu.CompilerParams(
            use_tc_tiling_on_sc=False, needs_layout_passes=False),
    )
    return call(emb_flat, lin_flat, idx_flat)


_BB = 2048


def _tc_body(fen_ref, lin_ref, den_ref, w1t_ref, b1_ref, w2t_ref, b2_ref,
             pt_ref, e_ref, s_ref, dw_ref, bias_ref, out_ref):
    x = fen_ref[...]
    h = jnp.dot(w1t_ref[...], x, preferred_element_type=jnp.float32)
    h = jnp.maximum(h + b1_ref[...], 0.0)
    h = jnp.dot(w2t_ref[...], h, preferred_element_type=jnp.float32)
    h = jnp.maximum(h + b2_ref[...], 0.0)
    logits = jnp.dot(pt_ref[...], h, preferred_element_type=jnp.float32)
    m = jnp.max(logits, axis=0, keepdims=True)
    e = jnp.exp(logits - m)
    mx = (float(F) / jnp.sum(e, axis=0, keepdims=True)) * e
    mx_exp = jnp.dot(e_ref[...], mx, preferred_element_type=jnp.float32)
    v = mx_exp * x
    sv = jnp.dot(s_ref[...], v, preferred_element_type=jnp.float32)
    fm = 0.5 * (jnp.sum(sv * sv, axis=0) - jnp.sum(v * v, axis=0))
    sp = jnp.sum(lin_ref[...] * mx, axis=0)
    dn = jnp.sum(den_ref[...] * dw_ref[...], axis=0)
    out_ref[...] = fm + sp + dn + bias_ref[0, 0]


def kernel(sparse, dense, emb, lin_w, dense_w, W1, b1, W2, b2, P, bias):
    emb_flat = emb.transpose(0, 2, 1).reshape(FD * V)
    lin_flat = lin_w.reshape(F * V)
    idx_flat = sparse.T.reshape(F * B)

    fen_flat, lin_g = _sc_gather(emb_flat, lin_flat, idx_flat)
    fen_t = fen_flat.reshape(FD, B)
    lin_t = lin_g.reshape(F, B)
    dense_t = dense.T

    expand_t = jnp.repeat(jnp.eye(F, dtype=jnp.float32), D, axis=0)
    reduce_t = jnp.tile(jnp.eye(D, dtype=jnp.float32), (1, F))

    out = pl.pallas_call(
        _tc_body,
        grid=(B // _BB,),
        in_specs=[
            pl.BlockSpec((FD, _BB), lambda i: (0, i)),
            pl.BlockSpec((F, _BB), lambda i: (0, i)),
            pl.BlockSpec((ND, _BB), lambda i: (0, i)),
            pl.BlockSpec((H1, FD), lambda i: (0, 0)),
            pl.BlockSpec((H1, 1), lambda i: (0, 0)),
            pl.BlockSpec((H2, H1), lambda i: (0, 0)),
            pl.BlockSpec((H2, 1), lambda i: (0, 0)),
            pl.BlockSpec((F, H2), lambda i: (0, 0)),
            pl.BlockSpec((FD, F), lambda i: (0, 0)),
            pl.BlockSpec((D, FD), lambda i: (0, 0)),
            pl.BlockSpec((ND, 1), lambda i: (0, 0)),
            pl.BlockSpec((1, 1), lambda i: (0, 0)),
        ],
        out_specs=pl.BlockSpec((_BB,), lambda i: (i,)),
        out_shape=jax.ShapeDtypeStruct((B,), jnp.float32),
    )(fen_t, lin_t, dense_t, W1.T, b1.reshape(H1, 1), W2.T, b2.reshape(H2, 1),
      P.T, expand_t, reduce_t, dense_w.reshape(ND, 1), bias.reshape(1, 1))
    return out


# speedup vs baseline: 1.6119x; 1.6119x over previous
import functools

import jax
import jax.numpy as jnp
from jax import lax
from jax.experimental import pallas as pl
from jax.experimental.pallas import tpu as pltpu
from jax.experimental.pallas import tpu_sc as plsc

B = 16384
F = 26
V = 100000
D = 16
ND = 13
H1 = 256
H2 = 128
FD = F * D

_NC = 2
_NS = 16
_NW = _NC * _NS
_NTASK = FD + F
_NROUND = -(-_NTASK // _NW)


_CH = 4096


def _sc_gather_body(emb_hbm, lin_hbm, idx_hbm, fen_out, lin_out,
                    row_v, idx_v, res_v):
    wid = lax.axis_index("s") * _NC + lax.axis_index("c")

    for j in range(_NROUND):
        t = jnp.minimum(wid + _NW * j, _NTASK - 1)
        is_lin = t >= FD
        f = jnp.where(is_lin, t - FD, t // D)
        pltpu.sync_copy(idx_hbm.at[pl.ds(f * B, B)], idx_v)

        @pl.when(jnp.logical_not(is_lin))
        def _le():
            pltpu.sync_copy(emb_hbm.at[pl.ds(t * V, V)], row_v)

        @pl.when(is_lin)
        def _ll():
            pltpu.sync_copy(lin_hbm.at[pl.ds((t - FD) * V, V)], row_v)

        for cix in range(B // _CH):
            @plsc.parallel_loop(0, _CH // 16, unroll=8)
            def _gather(i):
                off = i * 16
                ii = idx_v[pl.ds(cix * _CH + off, 16)]
                res_v[pl.ds(off, 16)] = plsc.load_gather(row_v, [ii])

            @pl.when(jnp.logical_not(is_lin))
            def _of():
                pltpu.sync_copy(
                    res_v, fen_out.at[pl.ds(t * B + cix * _CH, _CH)])

            @pl.when(is_lin)
            def _ol():
                pltpu.sync_copy(
                    res_v, lin_out.at[pl.ds((t - FD) * B + cix * _CH, _CH)])


def _sc_gather(emb_flat, lin_flat, idx_flat):
    call = pl.kernel(
        _sc_gather_body,
        out_type=(
            jax.ShapeDtypeStruct((FD * B,), jnp.float32),
            jax.ShapeDtypeStruct((F * B,), jnp.float32),
        ),
        mesh=plsc.VectorSubcoreMesh(core_axis_name="c", subcore_axis_name="s"),
        scratch_types=[
            pltpu.VMEM((V,), jnp.float32),
            pltpu.VMEM((B,), jnp.int32),
            pltpu.VMEM((_CH,), jnp.float32),
        ],
        compiler_params=pltpu.CompilerParams(
            use_tc_tiling_on_sc=False, needs_layout_passes=False),
    )
    return call(emb_flat, lin_flat, idx_flat)


_BB = 2048


def _tc_body(fen_ref, lin_ref, den_ref, w1t_ref, b1_ref, w2t_ref, b2_ref,
             pt_ref, e_ref, s_ref, dw_ref, bias_ref, out_ref):
    x = fen_ref[...]
    h = jnp.dot(w1t_ref[...], x, preferred_element_type=jnp.float32)
    h = jnp.maximum(h + b1_ref[...], 0.0)
    h = jnp.dot(w2t_ref[...], h, preferred_element_type=jnp.float32)
    h = jnp.maximum(h + b2_ref[...], 0.0)
    logits = jnp.dot(pt_ref[...], h, preferred_element_type=jnp.float32)
    m = jnp.max(logits, axis=0, keepdims=True)
    e = jnp.exp(logits - m)
    mx = (float(F) / jnp.sum(e, axis=0, keepdims=True)) * e
    mx_exp = jnp.dot(e_ref[...], mx, preferred_element_type=jnp.float32)
    v = mx_exp * x
    sv = jnp.dot(s_ref[...], v, preferred_element_type=jnp.float32)
    fm = 0.5 * (jnp.sum(sv * sv, axis=0) - jnp.sum(v * v, axis=0))
    sp = jnp.sum(lin_ref[...] * mx, axis=0)
    dn = jnp.sum(den_ref[...] * dw_ref[...], axis=0)
    out_ref[...] = fm + sp + dn + bias_ref[0, 0]


def kernel(sparse, dense, emb, lin_w, dense_w, W1, b1, W2, b2, P, bias):
    emb_flat = emb.transpose(0, 2, 1).reshape(FD * V)
    lin_flat = lin_w.reshape(F * V)
    idx_flat = sparse.T.reshape(F * B)

    fen_flat, lin_g = _sc_gather(emb_flat, lin_flat, idx_flat)
    fen_t = fen_flat.reshape(FD, B)
    lin_t = lin_g.reshape(F, B)
    dense_t = dense.T

    expand_t = jnp.repeat(jnp.eye(F, dtype=jnp.float32), D, axis=0)
    reduce_t = jnp.tile(jnp.eye(D, dtype=jnp.float32), (1, F))

    out = pl.pallas_call(
        _tc_body,
        grid=(B // _BB,),
        in_specs=[
            pl.BlockSpec((FD, _BB), lambda i: (0, i)),
            pl.BlockSpec((F, _BB), lambda i: (0, i)),
            pl.BlockSpec((ND, _BB), lambda i: (0, i)),
            pl.BlockSpec((H1, FD), lambda i: (0, 0)),
            pl.BlockSpec((H1, 1), lambda i: (0, 0)),
            pl.BlockSpec((H2, H1), lambda i: (0, 0)),
            pl.BlockSpec((H2, 1), lambda i: (0, 0)),
            pl.BlockSpec((F, H2), lambda i: (0, 0)),
            pl.BlockSpec((FD, F), lambda i: (0, 0)),
            pl.BlockSpec((D, FD), lambda i: (0, 0)),
            pl.BlockSpec((ND, 1), lambda i: (0, 0)),
            pl.BlockSpec((1, 1), lambda i: (0, 0)),
        ],
        out_specs=pl.BlockSpec((_BB,), lambda i: (i,)),
        out_shape=jax.ShapeDtypeStruct((B,), jnp.float32),
    )(fen_t, lin_t, dense_t, W1.T, b1.reshape(H1, 1), W2.T, b2.reshape(H2, 1),
      P.T, expand_t, reduce_t, dense_w.reshape(ND, 1), bias.reshape(1, 1))
    return out
